# Initial kernel scaffold; baseline (speedup 1.0000x reference)
import jax
import jax.numpy as jnp
from jax.experimental import pallas as pl


def kernel(inputs, token_table, position_table):
    raise NotImplementedError("write your pallas kernel here")


# speedup vs baseline: 1.3892x; 1.3892x over previous
import functools

import jax
import jax.numpy as jnp
from jax import lax
from jax.experimental import pallas as pl
from jax.experimental.pallas import tpu as pltpu
from jax.experimental.pallas import tpu_sc as plsc

VOCAB_SIZE = 1000000
SEQ_LEN = 200
EMBED_DIM = 32
BATCH = 4096

NC = 2
NS = 16
NW = NC * NS

B_FLAT = BATCH * SEQ_LEN
ROWS_PER_W = B_FLAT // NW
G = 100
CG = 8
C = CG * G
NCHUNK = ROWS_PER_W // C
REPS = C // SEQ_LEN


def _sc_body(idx_hbm, tok_hbm, pos_hbm, out_hbm, idx_v, rows_v, pos_v, sem):
    wid = lax.axis_index("s") * NC + lax.axis_index("c")
    base = wid * ROWS_PER_W

    pltpu.sync_copy(pos_hbm, pos_v)

    @pl.loop(0, NCHUNK)
    def _chunk(c):
        chunk_base = base + c * C
        pltpu.sync_copy(idx_hbm.at[wid * NCHUNK + c], idx_v)
        descs = []
        for j in range(CG):
            descs.append(
                pltpu.async_copy(
                    tok_hbm.at[idx_v.at[j]], rows_v.at[pl.ds(j * G, G)], sem
                )
            )
        for d in descs:
            d.wait()

        @pl.loop(0, SEQ_LEN)
        def _add(s):
            p0 = pos_v[s, 0:16]
            p1 = pos_v[s, 16:32]
            for r in range(REPS):
                rows_v[r * SEQ_LEN + s, 0:16] += p0
                rows_v[r * SEQ_LEN + s, 16:32] += p1

        pltpu.sync_copy(rows_v, out_hbm.at[pl.ds(chunk_base, C)])


@jax.jit
def _sc_embed(idx, token_table, position_table):
    mesh = plsc.VectorSubcoreMesh(
        core_axis_name="c", subcore_axis_name="s", num_cores=NC, num_subcores=NS
    )
    return pl.kernel(
        _sc_body,
        out_type=jax.ShapeDtypeStruct((B_FLAT, EMBED_DIM), jnp.float32),
        mesh=mesh,
        scratch_types=[
            pltpu.VMEM((CG, G), jnp.int32),
            pltpu.VMEM((C, EMBED_DIM), jnp.float32),
            pltpu.VMEM((SEQ_LEN, EMBED_DIM), jnp.float32),
            pltpu.SemaphoreType.DMA,
        ],
        compiler_params=pltpu.CompilerParams(use_tc_tiling_on_sc=False),
    )(idx, token_table, position_table)


def kernel(inputs, token_table, position_table):
    idx = inputs.astype(jnp.int32).reshape(NW * NCHUNK, CG, G)
    out = _sc_embed(idx, token_table, position_table)
    return out.reshape(BATCH, SEQ_LEN, EMBED_DIM)
